# Initial kernel scaffold; baseline (speedup 1.0000x reference)
import jax
import jax.numpy as jnp
from jax.experimental import pallas as pl


def kernel(xyz1, xyz2, features1, features2, W1, b1, gamma, beta, W2, b2):
    raise NotImplementedError("write your pallas kernel here")


# speedup vs baseline: 43.1710x; 43.1710x over previous
import functools

import jax
import jax.numpy as jnp
from jax import lax
from jax.experimental import pallas as pl
from jax.experimental.pallas import tpu as pltpu

BN = 512


def _fused_body(xyz1_ref, xyz2_ref, f1_ref, f2_ref, w1a_ref, w1b_ref,
                b1_ref, gamma_ref, beta_ref, w2_ref, b2_ref, out_ref):
    x = xyz1_ref[0]
    y = xyz2_ref[0]
    n2 = y.shape[0]

    xx = jnp.sum(x * x, axis=1)
    yy = jnp.sum(y * y, axis=1)
    xy = lax.dot_general(x, y, (((1,), (1,)), ((), ())),
                         preferred_element_type=jnp.float32)
    dists = xx[:, None] + yy[None, :] - 2.0 * xy

    iota = lax.broadcasted_iota(jnp.int32, dists.shape, 1)
    big = jnp.float32(3.4e38)
    d = dists
    wmat = jnp.zeros(dists.shape, jnp.float32)
    recips = []
    masks = []
    for _ in range(3):
        dk = jnp.min(d, axis=1)
        ik = jnp.min(jnp.where(d == dk[:, None], iota, n2), axis=1)
        mk = iota == ik[:, None]
        masks.append(mk)
        recips.append(1.0 / (dk + 1e-8))
        d = jnp.where(mk, big, d)
    norm = recips[0] + recips[1] + recips[2]
    for mk, rk in zip(masks, recips):
        wmat = wmat + jnp.where(mk, (rk / norm)[:, None], 0.0)

    interp = lax.dot_general(wmat, f2_ref[0], (((1,), (0,)), ((), ())),
                             preferred_element_type=jnp.float32)

    h = (lax.dot_general(f1_ref[0], w1a_ref[...], (((1,), (0,)), ((), ())),
                         preferred_element_type=jnp.float32)
         + lax.dot_general(interp, w1b_ref[...], (((1,), (0,)), ((), ())),
                           preferred_element_type=jnp.float32)
         + b1_ref[...])
    mu = jnp.mean(h, axis=1, keepdims=True)
    xc = h - mu
    var = jnp.mean(xc * xc, axis=1, keepdims=True)
    h = xc * lax.rsqrt(var + 1e-5) * gamma_ref[...] + beta_ref[...]
    h = jnp.maximum(h, 0.0)
    out_ref[0] = (lax.dot_general(h, w2_ref[...], (((1,), (0,)), ((), ())),
                                  preferred_element_type=jnp.float32)
                  + b2_ref[...])


@functools.partial(jax.jit, static_argnames=("interpret",))
def kernel(xyz1, xyz2, features1, features2, W1, b1, gamma, beta, W2, b2,
           interpret=False):
    B, N1, _ = xyz1.shape
    _, N2, _ = xyz2.shape
    Cskip = features1.shape[-1]
    Cin = features2.shape[-1]
    Cout = W2.shape[-1]
    W1a = W1[:Cskip]
    W1b = W1[Cskip:]
    b1r = b1.reshape(1, -1)
    gammar = gamma.reshape(1, -1)
    betar = beta.reshape(1, -1)
    b2r = b2.reshape(1, -1)

    grid = (B, N1 // BN)
    const = lambda shape: pl.BlockSpec(shape, lambda b, i: (0,) * len(shape))
    out = pl.pallas_call(
        _fused_body,
        grid=grid,
        in_specs=[
            pl.BlockSpec((1, BN, 3), lambda b, i: (b, i, 0)),
            pl.BlockSpec((1, N2, 3), lambda b, i: (b, 0, 0)),
            pl.BlockSpec((1, BN, Cskip), lambda b, i: (b, i, 0)),
            pl.BlockSpec((1, N2, Cin), lambda b, i: (b, 0, 0)),
            const((Cskip, Cout)),
            const((Cin, Cout)),
            const((1, Cout)),
            const((1, Cout)),
            const((1, Cout)),
            const((Cout, Cout)),
            const((1, Cout)),
        ],
        out_specs=pl.BlockSpec((1, BN, Cout), lambda b, i: (b, i, 0)),
        out_shape=jax.ShapeDtypeStruct((B, N1, Cout), jnp.float32),
        interpret=interpret,
    )(xyz1, xyz2, features1, features2, W1a, W1b, b1r, gammar, betar, W2, b2r)
    return out
